# baseline (speedup 1.0000x reference)
import jax
import jax.numpy as jnp
from jax import lax
from jax.experimental import pallas as pl
from jax.experimental.pallas import tpu as pltpu
from jax.experimental.pallas import tpu_sc as plsc

B, L, V, D = 16384, 26, 100000, 64
IN_DIM = L * D
T = L // 2
S = T * B

_info = plsc.get_sparse_core_info()
NC, NS = _info.num_cores, _info.num_subcores
NW = NC * NS
PER_W = S // NW
CHUNK = 416
N2 = PER_W // (2 * CHUNK)


def _sc_gather_body(
    table_hbm, ga_hbm, gb_hbm, out_hbm,
    ia_v, ib_v, ra0_v, rb0_v, ra1_v, rb1_v, sem0, sem1,
):
    wid = lax.axis_index("s") * NC + lax.axis_index("c")
    base = wid * PER_W

    pltpu.sync_copy(ga_hbm.at[pl.ds(base, PER_W)], ia_v)
    pltpu.sync_copy(gb_hbm.at[pl.ds(base, PER_W)], ib_v)

    def start(c, ra, rb, sem):
        off = c * CHUNK
        pltpu.async_copy(table_hbm.at[ia_v.at[pl.ds(off, CHUNK)]], ra, sem)
        pltpu.async_copy(table_hbm.at[ib_v.at[pl.ds(off, CHUNK)]], rb, sem)

    def drain(c, ra, rb, sem):
        pltpu.make_async_copy(table_hbm.at[ia_v.at[pl.ds(0, CHUNK)]], ra, sem).wait()
        pltpu.make_async_copy(table_hbm.at[ib_v.at[pl.ds(0, CHUNK)]], rb, sem).wait()
        row = base + c * CHUNK
        pltpu.sync_copy(ra, out_hbm.at[pl.ds(row, CHUNK), pl.ds(0, D)])
        pltpu.sync_copy(rb, out_hbm.at[pl.ds(row, CHUNK), pl.ds(D, D)])

    start(0, ra0_v, rb0_v, sem0)

    def step(i2, _):
        start(2 * i2 + 1, ra1_v, rb1_v, sem1)
        drain(2 * i2, ra0_v, rb0_v, sem0)

        @pl.when(i2 < N2 - 1)
        def _():
            start(2 * i2 + 2, ra0_v, rb0_v, sem0)

        drain(2 * i2 + 1, ra1_v, rb1_v, sem1)
        return _

    lax.fori_loop(0, N2, step, None)


def _sc_gather(table, ga, gb):
    return pl.kernel(
        _sc_gather_body,
        out_type=jax.ShapeDtypeStruct((S, 2 * D), jnp.float32),
        mesh=plsc.VectorSubcoreMesh(core_axis_name="c", subcore_axis_name="s"),
        scratch_types=[
            pltpu.VMEM((PER_W,), jnp.int32),
            pltpu.VMEM((PER_W,), jnp.int32),
            pltpu.VMEM((CHUNK, D), jnp.float32),
            pltpu.VMEM((CHUNK, D), jnp.float32),
            pltpu.VMEM((CHUNK, D), jnp.float32),
            pltpu.VMEM((CHUNK, D), jnp.float32),
            pltpu.SemaphoreType.DMA,
            pltpu.SemaphoreType.DMA,
        ],
        compiler_params=pltpu.CompilerParams(use_tc_tiling_on_sc=False),
    )(table, ga, gb)


R_BLK = 1024


def _mlp_body(x_ref, w1_ref, b1_ref, w2_ref, b2_ref, w3_ref, b3_ref, o_ref):
    h = jnp.dot(x_ref[0], w1_ref[0], preferred_element_type=jnp.float32)
    for t in range(1, T):
        h += jnp.dot(x_ref[t], w1_ref[t], preferred_element_type=jnp.float32)
    h = jnp.maximum(h + b1_ref[...], 0.0)
    h = jnp.dot(h, w2_ref[...], preferred_element_type=jnp.float32)
    h = jnp.maximum(h + b2_ref[...], 0.0)
    o_ref[...] = (
        jnp.dot(h, w3_ref[...], preferred_element_type=jnp.float32) + b3_ref[...]
    )


def _tc_mlp(x3, W1p, b1, W2, b2, W3, b3):
    grid = (B // R_BLK,)
    return pl.pallas_call(
        _mlp_body,
        grid=grid,
        in_specs=[
            pl.BlockSpec((T, R_BLK, 2 * D), lambda i: (0, i, 0)),
            pl.BlockSpec((T, 2 * D, 64), lambda i: (0, 0, 0)),
            pl.BlockSpec((1, 64), lambda i: (0, 0)),
            pl.BlockSpec((64, 32), lambda i: (0, 0)),
            pl.BlockSpec((1, 32), lambda i: (0, 0)),
            pl.BlockSpec((32, 1), lambda i: (0, 0)),
            pl.BlockSpec((1, 1), lambda i: (0, 0)),
        ],
        out_specs=pl.BlockSpec((R_BLK, 1), lambda i: (i, 0)),
        out_shape=jax.ShapeDtypeStruct((B, 1), jnp.float32),
    )(x3, W1p, b1.reshape(1, 64), W2, b2.reshape(1, 32), W3, b3.reshape(1, 1))


def kernel(genome_indices_batch, table, W1, b1, W2, b2, W3, b3):
    idx = genome_indices_batch.astype(jnp.int32)
    ga = idx[:, 0::2].T.reshape(-1)
    gb = idx[:, 1::2].T.reshape(-1)
    flat = _sc_gather(table, ga, gb)
    x3 = flat.reshape(T, B, 2 * D)
    return _tc_mlp(x3, W1.reshape(T, 2 * D, 64), b1, W2, b2, W3, b3)


# speedup vs baseline: 1.0115x; 1.0115x over previous
import jax
import jax.numpy as jnp
from jax import lax
from jax.experimental import pallas as pl
from jax.experimental.pallas import tpu as pltpu
from jax.experimental.pallas import tpu_sc as plsc

B, L, V, D = 16384, 26, 100000, 64
IN_DIM = L * D
T = L // 2
S = T * B

_info = plsc.get_sparse_core_info()
NC, NS = _info.num_cores, _info.num_subcores
NW = NC * NS
PER_W = S // NW
CHUNK = 416
N2 = PER_W // (2 * CHUNK)


def _sc_gather_body(
    table_hbm, ga_hbm, gb_hbm, out_hbm,
    ia_v, ib_v, ra0_v, rb0_v, ra1_v, rb1_v, sem0, sem1,
):
    wid = lax.axis_index("s") * NC + lax.axis_index("c")
    base = wid * PER_W

    pltpu.sync_copy(ga_hbm.at[pl.ds(base, PER_W)], ia_v)
    pltpu.sync_copy(gb_hbm.at[pl.ds(base, PER_W)], ib_v)

    def start(c, ra, rb, sem):
        off = c * CHUNK
        pltpu.async_copy(table_hbm.at[ia_v.at[pl.ds(off, CHUNK)]], ra, sem)
        pltpu.async_copy(table_hbm.at[ib_v.at[pl.ds(off, CHUNK)]], rb, sem)

    def drain(c, ra, rb, sem):
        pltpu.make_async_copy(table_hbm.at[ia_v.at[pl.ds(0, CHUNK)]], ra, sem).wait()
        pltpu.make_async_copy(table_hbm.at[ib_v.at[pl.ds(0, CHUNK)]], rb, sem).wait()
        row = base + c * CHUNK
        pltpu.sync_copy(ra, out_hbm.at[pl.ds(row, CHUNK), pl.ds(0, D)])
        pltpu.sync_copy(rb, out_hbm.at[pl.ds(row, CHUNK), pl.ds(D, D)])

    start(0, ra0_v, rb0_v, sem0)

    def step(i2, _):
        start(2 * i2 + 1, ra1_v, rb1_v, sem1)
        drain(2 * i2, ra0_v, rb0_v, sem0)

        @pl.when(i2 < N2 - 1)
        def _():
            start(2 * i2 + 2, ra0_v, rb0_v, sem0)

        drain(2 * i2 + 1, ra1_v, rb1_v, sem1)
        return _

    lax.fori_loop(0, N2, step, None)


def _sc_gather(table, ga, gb):
    return pl.kernel(
        _sc_gather_body,
        out_type=jax.ShapeDtypeStruct((S, 2 * D), jnp.float32),
        mesh=plsc.VectorSubcoreMesh(core_axis_name="c", subcore_axis_name="s"),
        scratch_types=[
            pltpu.VMEM((PER_W,), jnp.int32),
            pltpu.VMEM((PER_W,), jnp.int32),
            pltpu.VMEM((CHUNK, D), jnp.float32),
            pltpu.VMEM((CHUNK, D), jnp.float32),
            pltpu.VMEM((CHUNK, D), jnp.float32),
            pltpu.VMEM((CHUNK, D), jnp.float32),
            pltpu.SemaphoreType.DMA,
            pltpu.SemaphoreType.DMA,
        ],
        compiler_params=pltpu.CompilerParams(use_tc_tiling_on_sc=False),
    )(table, ga, gb)


R_BLK = 2048


def _mlp_body(x_ref, w1_ref, b1_ref, w2_ref, b2_ref, w3_ref, b3_ref, o_ref):
    h = jnp.dot(x_ref[0], w1_ref[0], preferred_element_type=jnp.float32)
    for t in range(1, T):
        h += jnp.dot(x_ref[t], w1_ref[t], preferred_element_type=jnp.float32)
    h = jnp.maximum(h + b1_ref[...], 0.0)
    h = jnp.dot(h, w2_ref[...], preferred_element_type=jnp.float32)
    h = jnp.maximum(h + b2_ref[...], 0.0)
    o_ref[...] = (
        jnp.dot(h, w3_ref[...], preferred_element_type=jnp.float32) + b3_ref[...]
    )


def _tc_mlp(x3, W1p, b1, W2, b2, W3, b3):
    grid = (B // R_BLK,)
    return pl.pallas_call(
        _mlp_body,
        grid=grid,
        in_specs=[
            pl.BlockSpec((T, R_BLK, 2 * D), lambda i: (0, i, 0)),
            pl.BlockSpec((T, 2 * D, 64), lambda i: (0, 0, 0)),
            pl.BlockSpec((1, 64), lambda i: (0, 0)),
            pl.BlockSpec((64, 32), lambda i: (0, 0)),
            pl.BlockSpec((1, 32), lambda i: (0, 0)),
            pl.BlockSpec((32, 1), lambda i: (0, 0)),
            pl.BlockSpec((1, 1), lambda i: (0, 0)),
        ],
        out_specs=pl.BlockSpec((R_BLK, 1), lambda i: (i, 0)),
        out_shape=jax.ShapeDtypeStruct((B, 1), jnp.float32),
    )(x3, W1p, b1.reshape(1, 64), W2, b2.reshape(1, 32), W3, b3.reshape(1, 1))


def kernel(genome_indices_batch, table, W1, b1, W2, b2, W3, b3):
    idx = genome_indices_batch.astype(jnp.int32)
    ga = idx[:, 0::2].T.reshape(-1)
    gb = idx[:, 1::2].T.reshape(-1)
    flat = _sc_gather(table, ga, gb)
    x3 = flat.reshape(T, B, 2 * D)
    return _tc_mlp(x3, W1.reshape(T, 2 * D, 64), b1, W2, b2, W3, b3)
